# baseline (speedup 1.0000x reference)
import jax
import jax.numpy as jnp
from jax import lax
from jax.experimental import pallas as pl
from jax.experimental.pallas import tpu as pltpu
from jax.experimental.pallas import tpu_sc as plsc

_P = 128
_ALPHA = 16
_LANES = 16
_NW = 32


def _sc_gather_call(W, tokens):
    Pdim, V = W.shape
    N = tokens.shape[0]
    rows_per = Pdim // _NW
    chunk = 4096
    while N % chunk:
        chunk //= 2
    n_chunks = N // chunk
    mesh = plsc.VectorSubcoreMesh(core_axis_name="c", subcore_axis_name="s",
                                  num_cores=2, num_subcores=16)

    def body(W_hbm, tok_hbm, out_hbm, wrow_v, tok_v, obuf_v,
             sem_tok, sem_row, sem_o0, sem_o1):
        cid = lax.axis_index("c")
        sid = lax.axis_index("s")
        wid = sid * 2 + cid
        pltpu.async_copy(tok_hbm, tok_v, sem_tok).wait()
        out_sems = (sem_o0, sem_o1)
        last = [None, None]
        k = 0
        for r in range(rows_per):
            p = wid * rows_per + r
            pltpu.async_copy(W_hbm.at[p], wrow_v, sem_row).wait()
            for c in range(n_chunks):
                b = k % 2
                if last[b] is not None:
                    last[b].wait()

                @plsc.parallel_loop(0, chunk // _LANES, unroll=8)
                def _gather(i, c=c, b=b):
                    idx = tok_v[pl.ds(c * chunk + i * _LANES, _LANES)]
                    obuf_v[b, pl.ds(i * _LANES, _LANES)] = plsc.load_gather(
                        wrow_v, [idx])

                last[b] = pltpu.async_copy(
                    obuf_v.at[b], out_hbm.at[p, pl.ds(c * chunk, chunk)],
                    out_sems[b])
                k += 1
        for d in last:
            if d is not None:
                d.wait()

    return pl.kernel(
        body,
        out_type=jax.ShapeDtypeStruct((Pdim, N), jnp.float32),
        mesh=mesh,
        scratch_types=[
            pltpu.VMEM((V,), jnp.float32),
            pltpu.VMEM((N,), jnp.int32),
            pltpu.VMEM((2, chunk), jnp.float32),
            pltpu.SemaphoreType.DMA,
            pltpu.SemaphoreType.DMA,
            pltpu.SemaphoreType.DMA,
            pltpu.SemaphoreType.DMA,
        ],
        compiler_params=pltpu.CompilerParams(needs_layout_passes=False),
    )(W, tokens)


def _tc_expand_call(UT, SP2, fT, B, T, Tc, t_base, prev=None):
    Pdim = UT.shape[0]
    N = B * T

    def body(ut_ref, sp_ref, ft_ref, *rest):
        u_ref, fo_ref, y_ref = rest[-3:]
        u = ut_ref[...].T
        sp = sp_ref[...]
        msk = u >= sp
        u_ref[...] = u
        fo_ref[...] = msk.astype(jnp.float32)
        t2 = jnp.where(msk, u, 0.0)
        ft = ft_ref[...]
        y = (jnp.broadcast_to(t2[:, None, :], (B, _ALPHA, Pdim))
             * jnp.broadcast_to(ft[None, :, :], (B, _ALPHA, Pdim)))
        y_ref[...] = y

    in_specs = [
        pl.BlockSpec((Pdim, B), lambda i: (0, i)),
        pl.BlockSpec((1, Pdim), lambda i: (0, 0)),
        pl.BlockSpec((_ALPHA, Pdim), lambda i: (0, 0)),
    ]
    operands = [UT, SP2, fT]
    aliases = {}
    if prev is not None:
        in_specs += [pl.BlockSpec(memory_space=pl.ANY)] * 3
        operands += list(prev)
        aliases = {3: 0, 4: 1, 5: 2}

    return pl.pallas_call(
        body,
        grid=(Tc,),
        in_specs=in_specs,
        out_specs=[
            pl.BlockSpec((B, Pdim), lambda i: (t_base + i, 0)),
            pl.BlockSpec((B, Pdim), lambda i: (t_base + i, 0)),
            pl.BlockSpec((B, _ALPHA, Pdim), lambda i: (0, t_base + i, 0)),
        ],
        out_shape=[
            jax.ShapeDtypeStruct((N, Pdim), jnp.float32),
            jax.ShapeDtypeStruct((N, Pdim), jnp.float32),
            jax.ShapeDtypeStruct((B, T * _ALPHA, Pdim), jnp.float32),
        ],
        input_output_aliases=aliases,
        compiler_params=pltpu.CompilerParams(
            dimension_semantics=("arbitrary",)),
    )(*operands)


def kernel(input_ids, W, f, SP):
    B, T = input_ids.shape
    tokens_m = input_ids.T.reshape(-1)
    T1 = T // 2
    sp2 = SP.reshape(1, _P).astype(jnp.float32)
    ft = f.T
    UT1 = _sc_gather_call(W, tokens_m[: T1 * B])
    UT2 = _sc_gather_call(W, tokens_m[T1 * B:])
    part1 = _tc_expand_call(UT1, sp2, ft, B, T, T1, 0)
    Um, Fm, Y4 = _tc_expand_call(UT2, sp2, ft, B, T, T - T1, T1, prev=part1)
    U = jnp.transpose(Um.reshape(T, B, _P), (1, 0, 2))
    F = jnp.transpose(Fm.reshape(T, B, _P), (1, 0, 2))
    Y = jnp.transpose(Y4.reshape(B, T, _ALPHA, _P), (0, 1, 3, 2))
    return (U, F, Y)


# speedup vs baseline: 1.1227x; 1.1227x over previous
import jax
import jax.numpy as jnp
from jax import lax
from jax.experimental import pallas as pl
from jax.experimental.pallas import tpu as pltpu
from jax.experimental.pallas import tpu_sc as plsc

_P = 128
_ALPHA = 16
_LANES = 16
_NW = 32


def _sc_gather_call(W, tokens):
    Pdim, V = W.shape
    N = tokens.shape[0]
    rows_per = Pdim // _NW
    chunk = 4096
    while N % chunk:
        chunk //= 2
    n_chunks = N // chunk
    mesh = plsc.VectorSubcoreMesh(core_axis_name="c", subcore_axis_name="s",
                                  num_cores=2, num_subcores=16)

    def body(W_hbm, tok_hbm, out_hbm, wrow_v, tok_v, obuf_v,
             sem_tok, sem_row, sem_o0, sem_o1):
        cid = lax.axis_index("c")
        sid = lax.axis_index("s")
        wid = sid * 2 + cid
        pltpu.async_copy(tok_hbm, tok_v, sem_tok).wait()
        out_sems = (sem_o0, sem_o1)
        last = [None, None]
        k = 0
        for r in range(rows_per):
            p = wid * rows_per + r
            pltpu.async_copy(W_hbm.at[p], wrow_v, sem_row).wait()
            for c in range(n_chunks):
                b = k % 2
                if last[b] is not None:
                    last[b].wait()

                @plsc.parallel_loop(0, chunk // _LANES, unroll=8)
                def _gather(i, c=c, b=b):
                    idx = tok_v[pl.ds(c * chunk + i * _LANES, _LANES)]
                    obuf_v[b, pl.ds(i * _LANES, _LANES)] = plsc.load_gather(
                        wrow_v, [idx])

                last[b] = pltpu.async_copy(
                    obuf_v.at[b], out_hbm.at[p, pl.ds(c * chunk, chunk)],
                    out_sems[b])
                k += 1
        for d in last:
            if d is not None:
                d.wait()

    return pl.kernel(
        body,
        out_type=jax.ShapeDtypeStruct((Pdim, N), jnp.float32),
        mesh=mesh,
        scratch_types=[
            pltpu.VMEM((V,), jnp.float32),
            pltpu.VMEM((N,), jnp.int32),
            pltpu.VMEM((2, chunk), jnp.float32),
            pltpu.SemaphoreType.DMA,
            pltpu.SemaphoreType.DMA,
            pltpu.SemaphoreType.DMA,
            pltpu.SemaphoreType.DMA,
        ],
        compiler_params=pltpu.CompilerParams(needs_layout_passes=False),
    )(W, tokens)


def _tc_expand_call(UT, SP2, fT, B, T, Tc, t_base, prev=None):
    Pdim = UT.shape[0]
    N = B * T

    G = 2 if Tc % 2 == 0 else 1

    def body(ut_ref, sp_ref, ft_ref, *rest):
        u_ref, fo_ref, y_ref = rest[-3:]
        u = ut_ref[...].T
        sp = sp_ref[...]
        msk = u >= sp
        u_ref[...] = u
        fo_ref[...] = msk.astype(jnp.float32)
        t2 = jnp.where(msk, u, 0.0)
        ft = ft_ref[...]
        for g in range(G):
            tg = t2[g * B:(g + 1) * B, :]
            y_ref[:, g * _ALPHA:(g + 1) * _ALPHA, :] = (
                jnp.broadcast_to(tg[:, None, :], (B, _ALPHA, Pdim))
                * jnp.broadcast_to(ft[None, :, :], (B, _ALPHA, Pdim)))

    in_specs = [
        pl.BlockSpec((Pdim, G * B), lambda i: (0, i)),
        pl.BlockSpec((1, Pdim), lambda i: (0, 0)),
        pl.BlockSpec((_ALPHA, Pdim), lambda i: (0, 0)),
    ]
    operands = [UT, SP2, fT]
    aliases = {}
    if prev is not None:
        in_specs += [pl.BlockSpec(memory_space=pl.ANY)] * 3
        operands += list(prev)
        aliases = {3: 0, 4: 1, 5: 2}

    tb = t_base // G

    return pl.pallas_call(
        body,
        grid=(Tc // G,),
        in_specs=in_specs,
        out_specs=[
            pl.BlockSpec((G * B, Pdim), lambda i: (tb + i, 0)),
            pl.BlockSpec((G * B, Pdim), lambda i: (tb + i, 0)),
            pl.BlockSpec((B, G * _ALPHA, Pdim), lambda i: (0, tb + i, 0)),
        ],
        out_shape=[
            jax.ShapeDtypeStruct((N, Pdim), jnp.float32),
            jax.ShapeDtypeStruct((N, Pdim), jnp.float32),
            jax.ShapeDtypeStruct((B, T * _ALPHA, Pdim), jnp.float32),
        ],
        input_output_aliases=aliases,
        compiler_params=pltpu.CompilerParams(
            dimension_semantics=("arbitrary",)),
    )(*operands)


def kernel(input_ids, W, f, SP):
    B, T = input_ids.shape
    tokens_m = input_ids.T.reshape(-1)
    sp2 = SP.reshape(1, _P).astype(jnp.float32)
    ft = f.T
    UT = _sc_gather_call(W, tokens_m)
    Um, Fm, Y4 = _tc_expand_call(UT, sp2, ft, B, T, T, 0)
    U = jnp.transpose(Um.reshape(T, B, _P), (1, 0, 2))
    F = jnp.transpose(Fm.reshape(T, B, _P), (1, 0, 2))
    Y = jnp.transpose(Y4.reshape(B, T, _ALPHA, _P), (0, 1, 3, 2))
    return (U, F, Y)
